# baseline (speedup 1.0000x reference)
import jax
import jax.numpy as jnp
from jax import lax
from jax.experimental import pallas as pl
from jax.experimental.pallas import tpu as pltpu
from jax.experimental.pallas import tpu_sc as plsc

B = 4096
F = 26
V = 100000
D = 32
ND = 13
NDP = 16
H1 = 128
H2 = 128

NC = 2
NS = 16
NW = NC * NS
BPW = B // NW
CHUNK = 128
CPW = BPW * F // CHUNK
NROW = B * F // CHUNK
GW = 128


def _sc_emb_body(idx_hbm, emb_hbm, out_hbm, idx_v, big_v, sem_g, sem_w):
    wid = lax.axis_index("s") * NC + lax.axis_index("c")
    pltpu.sync_copy(idx_hbm.at[wid], idx_v)

    pltpu.async_copy(emb_hbm.at[idx_v.at[0]], big_v.at[0], sem_g)

    def body(j, carry):
        b = j % 2
        pltpu.make_async_copy(emb_hbm.at[idx_v.at[j]], big_v.at[b], sem_g).wait()

        @pl.when(j > 0)
        def _():
            pltpu.make_async_copy(big_v.at[1 - b], out_hbm.at[wid, j - 1], sem_w).wait()

        @pl.when(j < CPW - 1)
        def _():
            pltpu.async_copy(emb_hbm.at[idx_v.at[j + 1]], big_v.at[1 - b], sem_g)

        pltpu.async_copy(big_v.at[b], out_hbm.at[wid, j], sem_w)
        return carry

    lax.fori_loop(0, CPW, body, 0)
    pltpu.make_async_copy(big_v.at[(CPW - 1) % 2], out_hbm.at[wid, CPW - 1], sem_w).wait()


def _build_sc_emb():
    return pl.kernel(
        _sc_emb_body,
        out_type=jax.ShapeDtypeStruct((NW, CPW, CHUNK, GW), jnp.float32),
        mesh=plsc.VectorSubcoreMesh(core_axis_name="c", subcore_axis_name="s",
                                    num_cores=NC, num_subcores=NS),
        scratch_types=[
            pltpu.VMEM((CPW, CHUNK), jnp.int32),
            pltpu.VMEM((2, CHUNK, GW), jnp.float32),
            pltpu.SemaphoreType.DMA,
            pltpu.SemaphoreType.DMA,
        ],
    )


def _sc_lin_body(idx_hbm, lin_hbm, lin_out, idx_v, lin_v, sem_l):
    wid = lax.axis_index("s") * NC + lax.axis_index("c")
    row0 = wid * CPW
    pltpu.sync_copy(idx_hbm.at[pl.ds(row0, CPW)], idx_v)

    def fire(j, carry):
        pltpu.async_copy(lin_hbm.at[idx_v.at[j]], lin_v.at[j], sem_l)
        return carry

    lax.fori_loop(0, CPW, fire, 0)

    def drain(j, carry):
        pltpu.make_async_copy(lin_hbm.at[idx_v.at[j]], lin_v.at[j], sem_l).wait()
        return carry

    lax.fori_loop(0, CPW, drain, 0)
    pltpu.sync_copy(lin_v, lin_out.at[pl.ds(row0, CPW)])


def _build_sc_lin():
    return pl.kernel(
        _sc_lin_body,
        out_type=jax.ShapeDtypeStruct((NROW, CHUNK), jnp.float32),
        mesh=plsc.VectorSubcoreMesh(core_axis_name="c", subcore_axis_name="s",
                                    num_cores=NC, num_subcores=NS),
        compiler_params=pltpu.CompilerParams(use_tc_tiling_on_sc=False),
        scratch_types=[
            pltpu.VMEM((CPW, CHUNK), jnp.int32),
            pltpu.VMEM((CPW, CHUNK), jnp.float32),
            pltpu.SemaphoreType.DMA,
        ],
    )


def _tc_body(big_ref, p_ref, lin_ref, dense_ref, s_ref, w1e_ref, w1d_ref,
             w2_ref, wout_ref, wd_ref, out_ref):
    dense = dense_ref[...]
    parts = []
    for f in range(F):
        pf = p_ref[:, f:f + 1]
        gf = big_ref[:, f * GW:f * GW + D]
        for p in range(1, 4):
            xf = big_ref[:, f * GW + p * D:f * GW + (p + 1) * D]
            gf = jnp.where(pf == p, xf, gf)
        parts.append(gf)
    g = jnp.concatenate(parts, axis=1)
    h = jnp.dot(g, w1e_ref[...], preferred_element_type=jnp.float32)
    h = h + jnp.dot(dense, w1d_ref[...], preferred_element_type=jnp.float32)
    h = jnp.maximum(h, 0.0)
    h = jnp.maximum(jnp.dot(h, w2_ref[...], preferred_element_type=jnp.float32), 0.0)
    dnn = jnp.dot(h, wout_ref[...], preferred_element_type=jnp.float32)
    sum_e = jnp.dot(g, s_ref[...], preferred_element_type=jnp.float32)
    fm = 0.5 * (jnp.sum(sum_e * sum_e, axis=1, keepdims=True)
                - jnp.sum(g * g, axis=1, keepdims=True))
    lin = (jnp.sum(lin_ref[...], axis=1, keepdims=True)
           + jnp.dot(dense, wd_ref[...], preferred_element_type=jnp.float32))
    z = lin + fm + dnn
    out_ref[...] = 1.0 / (1.0 + jnp.exp(-z))


BT = 512

_tc_call = pl.pallas_call(
    _tc_body,
    grid=(B // BT,),
    in_specs=[
        pl.BlockSpec((BT, F * GW), lambda i: (i, 0)),
        pl.BlockSpec((BT, F), lambda i: (i, 0)),
        pl.BlockSpec((BT, F), lambda i: (i, 0)),
        pl.BlockSpec((BT, NDP), lambda i: (i, 0)),
        pl.BlockSpec((F * D, D), lambda i: (0, 0)),
        pl.BlockSpec((F * D, H1), lambda i: (0, 0)),
        pl.BlockSpec((NDP, H1), lambda i: (0, 0)),
        pl.BlockSpec((H1, H2), lambda i: (0, 0)),
        pl.BlockSpec((H2, 1), lambda i: (0, 0)),
        pl.BlockSpec((NDP, 1), lambda i: (0, 0)),
    ],
    out_specs=pl.BlockSpec((BT, 1), lambda i: (i, 0)),
    out_shape=jax.ShapeDtypeStruct((B, 1), jnp.float32),
)


def kernel(sparse_indices, dense_features, emb_table, linear_table,
           w_dense, W1, W2, W_out):
    offsets = jnp.arange(F, dtype=jnp.int32) * V
    flat_idx = sparse_indices.astype(jnp.int32) + offsets[None, :]
    idx4 = (flat_idx // 4).reshape(NW, CPW, CHUNK)
    pmat = flat_idx % 4
    emb2 = emb_table.reshape(F * V // 4, 4 * D)
    big = _build_sc_emb()(idx4, emb2).reshape(B, F * GW)
    lin_rows = _build_sc_lin()(flat_idx.reshape(NROW, CHUNK),
                               linear_table.reshape(F * V))
    lin2d = lin_rows.reshape(B, F)
    dense_pad = jnp.pad(dense_features, ((0, 0), (0, NDP - ND)))
    w1e = W1[:F * D]
    w1d = jnp.pad(W1[F * D:], ((0, NDP - ND), (0, 0)))
    wd = jnp.pad(w_dense, ((0, NDP - ND), (0, 0)))
    s = jnp.tile(jnp.eye(D, dtype=jnp.float32), (F, 1))
    return _tc_call(big, pmat, lin2d, dense_pad, s, w1e, w1d, W2, W_out, wd)
